# baseline (speedup 1.0000x reference)
import functools

import jax
import jax.numpy as jnp
from jax import lax
from jax.experimental import pallas as pl
from jax.experimental.pallas import tpu as pltpu
from jax.experimental.pallas import tpu_sc as plsc

L = 16
CHUNK = 128


def _gmf_body(nc, bpw, d, users_h, movies_h, ut_h, mt_h, wb_h, out_h,
              uidx, midx, urows, mrows, wbv, outv, sem):
    wid = lax.axis_index("s") * nc + lax.axis_index("c")

    pltpu.sync_copy(wb_h, wbv)
    pltpu.sync_copy(users_h.at[wid], uidx)
    pltpu.sync_copy(movies_h.at[wid], midx)

    nchunk = bpw // CHUNK
    copies = []
    for j in range(nchunk):
        dst = pl.ds(j * CHUNK, CHUNK)
        copies.append(pltpu.async_copy(ut_h.at[uidx.at[j]], urows.at[dst], sem))
        copies.append(pltpu.async_copy(mt_h.at[midx.at[j]], mrows.at[dst], sem))
    for c in copies:
        c.wait()

    wvecs = [wbv[pl.ds(i * L, L)] for i in range(d // L)]
    ws = [wvecs[k // L][k % L] for k in range(d)]
    bias = wbv[pl.ds(pl.multiple_of(d, L), L)][0]
    iota = lax.iota(jnp.int32, L)
    cols = [jnp.full((L,), k, jnp.int32) for k in range(d)]

    def g_body(g, carry):
        rid = g * L + iota
        acc = jnp.zeros((L,), jnp.float32)
        for k in range(d):
            gu = plsc.load_gather(urows, [rid, cols[k]])
            gm = plsc.load_gather(mrows, [rid, cols[k]])
            acc = acc + gu * gm * ws[k]
        x = acc + bias
        y = 1.0 / (1.0 + jnp.exp(-x))
        outv[pl.ds(pl.multiple_of(g * L, L), L)] = y
        return carry

    lax.fori_loop(0, bpw // L, g_body, 0)
    pltpu.sync_copy(outv, out_h.at[wid])


def kernel(users, movies, user_table, movie_table, W, b):
    batch = users.shape[0]
    d = user_table.shape[1]

    info = plsc.get_sparse_core_info()
    nc, ns = info.num_cores, info.num_subcores
    nw = nc * ns
    bpw = batch // nw

    users3 = users.astype(jnp.int32).reshape(nw, bpw // CHUNK, CHUNK)
    movies3 = movies.astype(jnp.int32).reshape(nw, bpw // CHUNK, CHUNK)
    wb = jnp.concatenate([W.reshape(-1), b.reshape(-1),
                          jnp.zeros((15,), jnp.float32)])

    mesh = plsc.VectorSubcoreMesh(core_axis_name="c", subcore_axis_name="s")
    run = pl.kernel(
        functools.partial(_gmf_body, nc, bpw, d),
        out_type=jax.ShapeDtypeStruct((nw, bpw), jnp.float32),
        mesh=mesh,
        compiler_params=pltpu.CompilerParams(needs_layout_passes=False,
                                             use_tc_tiling_on_sc=False),
        scratch_types=[
            pltpu.VMEM((bpw // CHUNK, CHUNK), jnp.int32),
            pltpu.VMEM((bpw // CHUNK, CHUNK), jnp.int32),
            pltpu.VMEM((bpw, d), jnp.float32),
            pltpu.VMEM((bpw, d), jnp.float32),
            pltpu.VMEM((d + 16,), jnp.float32),
            pltpu.VMEM((bpw,), jnp.float32),
            pltpu.SemaphoreType.DMA,
        ],
    )
    out = run(users3, movies3, user_table, movie_table, wb)
    return out.reshape(batch, 1)
